# baseline (speedup 1.0000x reference)
import functools

import jax
import jax.numpy as jnp
from jax.experimental import pallas as pl
from jax.experimental.pallas import tpu as pltpu


def _fused_cnn_kernel(bt, x_ref, w1t_ref, b1t_ref, w2t_ref, b2t_ref,
                      wf1_ref, bf1_ref, wf2_ref, bf2_ref, o_ref):
    f32 = jnp.float32

    zx = jnp.zeros((1, bt, 128), jnp.bfloat16)
    xh = jnp.concatenate(
        [zx, jnp.pad(x_ref[...], ((0, 0), (0, 0), (0, 32))), zx], axis=0)

    ch1 = max(2, min(16, 4096 // bt))
    a1s = []
    for hc in range(32 // ch1):
        taps = [xh[ch1 * hc + dh:ch1 * hc + dh + ch1] for dh in range(3)]
        lhs = jnp.concatenate(taps, axis=-1).reshape(ch1 * bt, 384)
        acc = jnp.dot(lhs, w1t_ref[...], preferred_element_type=f32)
        a = acc.reshape(ch1 // 2, 2, bt, 1024)
        m = jnp.maximum(
            jnp.maximum(a[:, 0, :, :512], a[:, 0, :, 512:]),
            jnp.maximum(a[:, 1, :, :512], a[:, 1, :, 512:]))
        a1s.append(jnp.maximum(m + b1t_ref[...], 0.0).astype(jnp.bfloat16))
    a1 = jnp.concatenate(a1s, axis=0)

    zc = jnp.zeros((16, bt, 32), jnp.bfloat16)
    s2 = jnp.concatenate([zc, a1, zc], axis=-1)
    zr = jnp.zeros((1, bt, 576), jnp.bfloat16)
    s2 = jnp.concatenate([zr, s2, zr], axis=0)

    ch2 = max(2, min(8, 2048 // bt))
    c2s = []
    for hc in range(16 // ch2):
        qs = []
        for q in range(4):
            taps = [jnp.pad(
                s2[ch2 * hc + dh:ch2 * hc + dh + ch2, :, 128 * q:128 * q + 192],
                ((0, 0), (0, 0), (0, 64))) for dh in range(3)]
            qs.append(jnp.concatenate(taps, axis=-1))
        lhs = jnp.stack(qs, axis=0).reshape(4 * ch2 * bt, 768)
        acc = jnp.dot(lhs, w2t_ref[...], preferred_element_type=f32)
        a = acc.reshape(4, ch2 // 2, 2, bt, 256)
        m = jnp.maximum(
            jnp.maximum(a[:, :, 0, :, :128], a[:, :, 0, :, 128:]),
            jnp.maximum(a[:, :, 1, :, :128], a[:, :, 1, :, 128:]))
        m = jnp.maximum(m + b2t_ref[...], 0.0)
        c2s.append(jnp.concatenate([m[q] for q in range(4)],
                                   axis=-1).astype(jnp.bfloat16))
    c2 = jnp.concatenate(c2s, axis=0)

    xf = jnp.concatenate([c2[h] for h in range(8)], axis=-1)
    h1 = jnp.dot(xf, wf1_ref[...], preferred_element_type=f32)
    h1 = jnp.maximum(h1 + bf1_ref[...], 0.0).astype(jnp.bfloat16)
    z = jnp.dot(h1, wf2_ref[...], preferred_element_type=f32) + bf2_ref[...]
    z = z - jnp.max(z, axis=-1, keepdims=True)
    e = jnp.exp(z)
    o_ref[...] = (e / jnp.sum(e, axis=-1, keepdims=True)).astype(o_ref.dtype)


def kernel(x_nchw, w1, b1, w2, b2, w_fc1, b_fc1, w_fc2, b_fc2):
    B = x_nchw.shape[0]
    bt = 128 if B % 128 == 0 else B

    xt = jnp.transpose(x_nchw, (2, 0, 1, 3))
    xr = xt.reshape(32, B, 96).astype(jnp.bfloat16)

    w1r = w1.reshape(3, 3, 3, 32)
    E1 = (jnp.arange(32)[None, :, None, None] + 1
          == 2 * jnp.arange(16)[None, None, None, :]
          + jnp.arange(2)[None, None, :, None]
          + jnp.arange(3)[:, None, None, None])
    w1t = jnp.einsum('kdpw,hkco->hcdpwo', E1.astype(w1.dtype), w1r)
    w1t = jnp.pad(w1t.reshape(3, 96, 1024), ((0, 0), (0, 32), (0, 0)))
    w1t = w1t.reshape(384, 1024).astype(jnp.bfloat16)

    w2r = w2.reshape(3, 3, 32, 64)
    E2 = (jnp.arange(6)[None, :, None, None]
          == 2 * jnp.arange(2)[None, None, None, :]
          + jnp.arange(2)[None, None, :, None]
          + jnp.arange(3)[:, None, None, None])
    w2t = jnp.einsum('kdpw,hkco->hdcpwo', E2.astype(w2.dtype), w2r)
    w2t = jnp.pad(w2t.reshape(3, 192, 256), ((0, 0), (0, 64), (0, 0)))
    w2t = w2t.reshape(768, 256).astype(jnp.bfloat16)

    b1t = jnp.tile(b1, (1, 16))
    b2t = jnp.tile(b2, (1, 2))
    wf1 = w_fc1.astype(jnp.bfloat16)
    wf2 = w_fc2.astype(jnp.bfloat16)

    kernel_fn = functools.partial(_fused_cnn_kernel, bt)
    return pl.pallas_call(
        kernel_fn,
        out_shape=jax.ShapeDtypeStruct((B, 100), jnp.float32),
        grid=(B // bt,),
        in_specs=[
            pl.BlockSpec((32, bt, 96), lambda i: (0, i, 0)),
            pl.BlockSpec((384, 1024), lambda i: (0, 0)),
            pl.BlockSpec((1, 512), lambda i: (0, 0)),
            pl.BlockSpec((768, 256), lambda i: (0, 0)),
            pl.BlockSpec((1, 128), lambda i: (0, 0)),
            pl.BlockSpec((4096, 512), lambda i: (0, 0)),
            pl.BlockSpec((1, 512), lambda i: (0, 0)),
            pl.BlockSpec((512, 100), lambda i: (0, 0)),
            pl.BlockSpec((1, 100), lambda i: (0, 0)),
        ],
        out_specs=pl.BlockSpec((bt, 100), lambda i: (i, 0)),
        compiler_params=pltpu.CompilerParams(
            dimension_semantics=("parallel",),
            vmem_limit_bytes=64 * 1024 * 1024),
    )(xr, w1t, b1t, w2t, b2t, wf1, b_fc1, wf2, b_fc2)


# speedup vs baseline: 1.0027x; 1.0027x over previous
import functools

import jax
import jax.numpy as jnp
from jax.experimental import pallas as pl
from jax.experimental.pallas import tpu as pltpu


def _fused_cnn_kernel(bt, x_ref, w1t_ref, b1t_ref, w2t_ref, b2t_ref,
                      wf1_ref, bf1_ref, wf2_ref, bf2_ref, o_ref):
    f32 = jnp.float32

    zx = jnp.zeros((1, bt, 128), jnp.bfloat16)
    xh = jnp.concatenate(
        [zx, jnp.pad(x_ref[...], ((0, 0), (0, 0), (0, 32))), zx], axis=0)

    ch1 = max(2, min(16, 4096 // bt))
    a1s = []
    for hc in range(32 // ch1):
        taps = [xh[ch1 * hc + dh:ch1 * hc + dh + ch1] for dh in range(3)]
        lhs = jnp.concatenate(taps, axis=-1).reshape(ch1 * bt, 384)
        acc = jnp.dot(lhs, w1t_ref[...], preferred_element_type=f32)
        a = acc.reshape(ch1 // 2, 2, bt, 1024)
        m = jnp.maximum(
            jnp.maximum(a[:, 0, :, :512], a[:, 0, :, 512:]),
            jnp.maximum(a[:, 1, :, :512], a[:, 1, :, 512:]))
        a1s.append(jnp.maximum(m + b1t_ref[...], 0.0).astype(jnp.bfloat16))
    a1 = jnp.concatenate(a1s, axis=0)

    zc = jnp.zeros((16, bt, 32), jnp.bfloat16)
    s2 = jnp.concatenate([zc, a1, zc], axis=-1)
    zr = jnp.zeros((1, bt, 576), jnp.bfloat16)
    s2 = jnp.concatenate([zr, s2, zr], axis=0)

    ch2 = max(2, min(8, 2048 // bt))
    c2s = []
    for hc in range(16 // ch2):
        qs = []
        for q in range(4):
            taps = [jnp.pad(
                s2[ch2 * hc + dh:ch2 * hc + dh + ch2, :, 128 * q:128 * q + 192],
                ((0, 0), (0, 0), (0, 64))) for dh in range(3)]
            qs.append(jnp.concatenate(taps, axis=-1))
        lhs = jnp.stack(qs, axis=0).reshape(4 * ch2 * bt, 768)
        acc = jnp.dot(lhs, w2t_ref[...], preferred_element_type=f32)
        a = acc.reshape(4, ch2 // 2, 2, bt, 256)
        m = jnp.maximum(
            jnp.maximum(a[:, :, 0, :, :128], a[:, :, 0, :, 128:]),
            jnp.maximum(a[:, :, 1, :, :128], a[:, :, 1, :, 128:]))
        m = jnp.maximum(m + b2t_ref[...], 0.0)
        c2s.append(jnp.concatenate([m[q] for q in range(4)],
                                   axis=-1).astype(jnp.bfloat16))
    c2 = jnp.concatenate(c2s, axis=0)

    xf = jnp.concatenate([c2[h] for h in range(8)], axis=-1)
    h1 = jnp.dot(xf, wf1_ref[...], preferred_element_type=f32)
    h1 = jnp.maximum(h1 + bf1_ref[...], 0.0).astype(jnp.bfloat16)
    z = jnp.dot(h1, wf2_ref[...], preferred_element_type=f32) + bf2_ref[...]
    z = z - jnp.max(z, axis=-1, keepdims=True)
    e = jnp.exp(z)
    o_ref[...] = (e / jnp.sum(e, axis=-1, keepdims=True)).astype(o_ref.dtype)


def kernel(x_nchw, w1, b1, w2, b2, w_fc1, b_fc1, w_fc2, b_fc2):
    B = x_nchw.shape[0]
    bt = 256 if B % 256 == 0 else B

    xt = jnp.transpose(x_nchw, (2, 0, 1, 3))
    xr = xt.reshape(32, B, 96).astype(jnp.bfloat16)

    w1r = w1.reshape(3, 3, 3, 32)
    E1 = (jnp.arange(32)[None, :, None, None] + 1
          == 2 * jnp.arange(16)[None, None, None, :]
          + jnp.arange(2)[None, None, :, None]
          + jnp.arange(3)[:, None, None, None])
    w1t = jnp.einsum('kdpw,hkco->hcdpwo', E1.astype(w1.dtype), w1r)
    w1t = jnp.pad(w1t.reshape(3, 96, 1024), ((0, 0), (0, 32), (0, 0)))
    w1t = w1t.reshape(384, 1024).astype(jnp.bfloat16)

    w2r = w2.reshape(3, 3, 32, 64)
    E2 = (jnp.arange(6)[None, :, None, None]
          == 2 * jnp.arange(2)[None, None, None, :]
          + jnp.arange(2)[None, None, :, None]
          + jnp.arange(3)[:, None, None, None])
    w2t = jnp.einsum('kdpw,hkco->hdcpwo', E2.astype(w2.dtype), w2r)
    w2t = jnp.pad(w2t.reshape(3, 192, 256), ((0, 0), (0, 64), (0, 0)))
    w2t = w2t.reshape(768, 256).astype(jnp.bfloat16)

    b1t = jnp.tile(b1, (1, 16))
    b2t = jnp.tile(b2, (1, 2))
    wf1 = w_fc1.astype(jnp.bfloat16)
    wf2 = w_fc2.astype(jnp.bfloat16)

    kernel_fn = functools.partial(_fused_cnn_kernel, bt)
    return pl.pallas_call(
        kernel_fn,
        out_shape=jax.ShapeDtypeStruct((B, 100), jnp.float32),
        grid=(B // bt,),
        in_specs=[
            pl.BlockSpec((32, bt, 96), lambda i: (0, i, 0)),
            pl.BlockSpec((384, 1024), lambda i: (0, 0)),
            pl.BlockSpec((1, 512), lambda i: (0, 0)),
            pl.BlockSpec((768, 256), lambda i: (0, 0)),
            pl.BlockSpec((1, 128), lambda i: (0, 0)),
            pl.BlockSpec((4096, 512), lambda i: (0, 0)),
            pl.BlockSpec((1, 512), lambda i: (0, 0)),
            pl.BlockSpec((512, 100), lambda i: (0, 0)),
            pl.BlockSpec((1, 100), lambda i: (0, 0)),
        ],
        out_specs=pl.BlockSpec((bt, 100), lambda i: (i, 0)),
        compiler_params=pltpu.CompilerParams(
            dimension_semantics=("parallel",),
            vmem_limit_bytes=64 * 1024 * 1024),
    )(xr, w1t, b1t, w2t, b2t, wf1, b_fc1, wf2, b_fc2)
